# baseline (speedup 1.0000x reference)
import functools

import jax
import jax.numpy as jnp
from jax import lax
from jax.experimental import pallas as pl
from jax.experimental.pallas import tpu as pltpu
from jax.experimental.pallas import tpu_sc as plsc

_K = 16
_NCLS = 10
_CHUNK = 512
_BT = 1024


def _knn_kernel(n_chunks, chunk,
                x_ref, w1_ref, b1_ref, w2_ref, b2_ref, dbt_ref,
                outv_ref, outi_ref,
                dist_s, vals_s, idx_s):
    bt = x_ref.shape[0]
    f32 = jnp.float32
    i32 = jnp.int32

    h = jnp.dot(x_ref[...], w1_ref[...], preferred_element_type=f32) + b1_ref[...]
    h = jnp.maximum(h, 0.0)
    h = jnp.dot(h, w2_ref[...], preferred_element_type=f32) + b2_ref[...]
    q = jax.nn.sigmoid(h)
    q2 = jnp.sum(q * q, axis=1, keepdims=True)

    vals_s[...] = jnp.full((bt, _K), jnp.inf, f32)
    idx_s[...] = jnp.full((bt, _K), 2 ** 30, i32)

    slot_i = jax.lax.broadcasted_iota(i32, (bt, _K), 1)
    lane_s = jax.lax.broadcasted_iota(i32, (bt, chunk), 1)

    def chunk_body(c, worst):
        db_c = dbt_ref[:, pl.ds(c * chunk, chunk)]
        d2c = 0.25 * jnp.sum(db_c * db_c, axis=0, keepdims=True)
        qd = jax.lax.dot_general(q, db_c, (((1,), (0,)), ((), ())),
                                 preferred_element_type=f32)
        dist = (q2 + d2c) + qd
        dist_s[...] = dist
        m0 = jnp.min(dist, axis=1, keepdims=True)

        def cond(st):
            return st[0]

        def body(st):
            _, m, worst = st
            d = dist_s[...]
            do = m < worst
            tmp = jnp.where(d == m, lane_s, chunk)
            amin = jnp.min(tmp, axis=1, keepdims=True)
            amin_g = jnp.where(do, amin, -1)
            gidx = c * chunk + amin
            d = jnp.where(tmp == amin_g, jnp.inf, d)
            dist_s[...] = d
            vals = vals_s[...]
            idxs = idx_s[...]
            wmax = jnp.max(vals, axis=1, keepdims=True)
            c1 = vals == wmax
            imax = jnp.max(jnp.where(c1, idxs, -1), axis=1, keepdims=True)
            c2 = c1 & (idxs == imax)
            smax = jnp.max(jnp.where(c2, slot_i, -1), axis=1, keepdims=True)
            pick = (slot_i == smax) & do
            vals_s[...] = jnp.where(pick, m, vals)
            idx_s[...] = jnp.where(pick, gidx, idxs)
            worst = jnp.max(vals_s[...], axis=1, keepdims=True)
            m = jnp.min(d, axis=1, keepdims=True)
            return jnp.any(m < worst), m, worst

        cont0 = jnp.any(m0 < worst)
        _, _, worst = jax.lax.while_loop(cond, body, (cont0, m0, worst))
        return worst

    worst0 = jnp.full((bt, 1), jnp.inf, f32)
    jax.lax.fori_loop(0, n_chunks, chunk_body, worst0)

    vals = vals_s[...]
    idxs = idx_s[...]
    sortedv = jnp.zeros((bt, _K), f32)
    for j in range(_K):
        mv = jnp.min(vals, axis=1, keepdims=True)
        c1 = vals == mv
        mi = jnp.min(jnp.where(c1, idxs, 2 ** 30), axis=1, keepdims=True)
        pick = c1 & (idxs == mi)
        sortedv = jnp.where(slot_i == j, -mv, sortedv)
        vals = jnp.where(pick, jnp.inf, vals)
    outv_ref[...] = sortedv
    outi_ref[...] = idxs


def _vote_kernel(b, n_db, idx3_hbm, dby_hbm, pred_hbm, dby_v, idx_v, pred_v):
    i32 = jnp.int32
    info = plsc.get_sparse_core_info()
    nc, ns = info.num_cores, info.num_subcores
    nw = nc * ns
    rows = b // nw
    wid = lax.axis_index("s") * nc + lax.axis_index("c")
    base = wid * rows
    pltpu.sync_copy(dby_hbm, dby_v)
    pltpu.sync_copy(idx3_hbm.at[wid], idx_v)
    for batch in range(rows // 16):
        cnt = [jnp.zeros((16,), i32) for _ in range(_NCLS)]
        for j in range(_K):
            ii = idx_v[j, pl.ds(batch * 16, 16)]
            labs = plsc.load_gather(dby_v, [ii])
            for cl in range(_NCLS):
                cnt[cl] = cnt[cl] + jnp.where(labs == cl, i32(1), i32(0))
        best_c = cnt[0]
        best_k = jnp.zeros((16,), i32)
        for cl in range(1, _NCLS):
            better = cnt[cl] > best_c
            best_c = jnp.where(better, cnt[cl], best_c)
            best_k = jnp.where(better, jnp.full((16,), cl, i32), best_k)
        pred_v[pl.ds(batch * 16, 16)] = best_k
    pltpu.sync_copy(pred_v, pred_hbm.at[pl.ds(base, rows)])


def kernel(x, conv_w, conv_b, lin_w, lin_b, db_x, db_y):
    b, _ = x.shape
    n_db = db_x.shape[0]
    chunk = _CHUNK
    n_chunks = -(-n_db // chunk)
    n_pad = n_chunks * chunk
    bt = min(_BT, b)

    w1 = conv_w[:, :, 3].T
    b1 = conv_b.reshape(1, -1)
    w2 = lin_w.T
    b2 = lin_b.reshape(1, -1)
    dbt = jnp.pad(-2.0 * db_x, ((0, n_pad - n_db), (0, 0)),
                  constant_values=1e18).T

    f32 = jnp.float32
    i32 = jnp.int32
    grid = (b // bt,)
    outv, outi = pl.pallas_call(
        lambda *a: _knn_kernel(n_chunks, chunk, *a),
        grid=grid,
        in_specs=[
            pl.BlockSpec((bt, x.shape[1]), lambda i: (i, 0)),
            pl.BlockSpec(w1.shape, lambda i: (0, 0)),
            pl.BlockSpec(b1.shape, lambda i: (0, 0)),
            pl.BlockSpec(w2.shape, lambda i: (0, 0)),
            pl.BlockSpec(b2.shape, lambda i: (0, 0)),
            pl.BlockSpec(dbt.shape, lambda i: (0, 0)),
        ],
        out_specs=[
            pl.BlockSpec((bt, _K), lambda i: (i, 0)),
            pl.BlockSpec((bt, _K), lambda i: (i, 0)),
        ],
        out_shape=[
            jax.ShapeDtypeStruct((b, _K), f32),
            jax.ShapeDtypeStruct((b, _K), i32),
        ],
        scratch_shapes=[
            pltpu.VMEM((bt, chunk), f32),
            pltpu.VMEM((bt, _K), f32),
            pltpu.VMEM((bt, _K), i32),
        ],
        compiler_params=pltpu.CompilerParams(
            dimension_semantics=("arbitrary",),
        ),
    )(x, w1, b1, w2, b2, dbt)

    dby = db_y.astype(i32)
    mesh = plsc.VectorSubcoreMesh(core_axis_name="c", subcore_axis_name="s")
    info = plsc.get_sparse_core_info()
    nw = info.num_cores * info.num_subcores
    rows = b // nw
    idx3 = outi.reshape(nw, rows, _K).transpose(0, 2, 1)
    vote = functools.partial(
        pl.kernel,
        mesh=mesh,
        out_type=jax.ShapeDtypeStruct((b,), i32),
        scratch_types=[
            pltpu.VMEM((n_db,), i32),
            pltpu.VMEM((_K, rows), i32),
            pltpu.VMEM((rows,), i32),
        ],
        compiler_params=pltpu.CompilerParams(needs_layout_passes=False),
    )(functools.partial(_vote_kernel, b, n_db))
    pred = vote(idx3, dby)
    return outv, pred


# speedup vs baseline: 1.1767x; 1.1767x over previous
import functools

import jax
import jax.numpy as jnp
from jax import lax
from jax.experimental import pallas as pl
from jax.experimental.pallas import tpu as pltpu
from jax.experimental.pallas import tpu_sc as plsc

_K = 16
_NCLS = 10
_CHUNK = 1024
_BT = 1024


def _knn_kernel(n_chunks, chunk,
                x_ref, w1_ref, b1_ref, w2_ref, b2_ref, dbt_ref,
                outv_ref, outi_ref,
                dist_s, vals_s, idx_s):
    bt = x_ref.shape[0]
    f32 = jnp.float32
    i32 = jnp.int32

    h = jnp.dot(x_ref[...], w1_ref[...], preferred_element_type=f32) + b1_ref[...]
    h = jnp.maximum(h, 0.0)
    h = jnp.dot(h, w2_ref[...], preferred_element_type=f32) + b2_ref[...]
    q = jax.nn.sigmoid(h)
    q2 = jnp.sum(q * q, axis=1, keepdims=True)

    vals_s[...] = jnp.full((bt, _K), jnp.inf, f32)
    idx_s[...] = jnp.full((bt, _K), 2 ** 30, i32)

    slot_i = jax.lax.broadcasted_iota(i32, (bt, _K), 1)
    lane_f = jax.lax.broadcasted_iota(i32, (bt, chunk), 1).astype(f32)

    def chunk_body(c, worst):
        db_c = dbt_ref[:, pl.ds(c * chunk, chunk)]
        d2c = 0.25 * jnp.sum(db_c * db_c, axis=0, keepdims=True)
        qd = jax.lax.dot_general(q, db_c, (((1,), (0,)), ((), ())),
                                 preferred_element_type=f32)
        dist = (q2 + d2c) + qd
        dist_s[...] = dist
        m0 = jnp.min(dist, axis=1, keepdims=True)

        def cond(st):
            return st[0]

        def body(st):
            _, m, worst = st
            d = dist_s[...]
            do = m < worst
            tmp = jnp.where(d == m, lane_f, float(chunk))
            amin = jnp.min(tmp, axis=1, keepdims=True)
            amin_g = jnp.where(do, amin, -1.0)
            gidx = c * chunk + amin.astype(i32)
            d = jnp.where(tmp == amin_g, jnp.inf, d)
            dist_s[...] = d
            vals = vals_s[...]
            idxs = idx_s[...]
            c1 = vals == worst
            imax = jnp.max(jnp.where(c1, idxs, -1), axis=1, keepdims=True)
            c2 = c1 & (idxs == imax)
            smax = jnp.max(jnp.where(c2, slot_i, -1), axis=1, keepdims=True)
            pick = (slot_i == smax) & do
            vals = jnp.where(pick, m, vals)
            idxs = jnp.where(pick, gidx, idxs)
            vals_s[...] = vals
            idx_s[...] = idxs
            worst = jnp.max(vals, axis=1, keepdims=True)
            m = jnp.min(d, axis=1, keepdims=True)
            return jnp.any(m < worst), m, worst

        cont0 = jnp.any(m0 < worst)
        _, _, worst = jax.lax.while_loop(cond, body, (cont0, m0, worst))
        return worst

    worst0 = jnp.full((bt, 1), jnp.inf, f32)
    jax.lax.fori_loop(0, n_chunks, chunk_body, worst0)

    vals = vals_s[...]
    idxs = idx_s[...]
    sortedv = jnp.zeros((bt, _K), f32)
    for j in range(_K):
        mv = jnp.min(vals, axis=1, keepdims=True)
        c1 = vals == mv
        mi = jnp.min(jnp.where(c1, idxs, 2 ** 30), axis=1, keepdims=True)
        pick = c1 & (idxs == mi)
        sortedv = jnp.where(slot_i == j, -mv, sortedv)
        vals = jnp.where(pick, jnp.inf, vals)
    outv_ref[...] = sortedv
    outi_ref[...] = idxs


def _vote_kernel(b, n_db, idx3_hbm, dby_hbm, pred_hbm, dby_v, idx_v, pred_v):
    i32 = jnp.int32
    info = plsc.get_sparse_core_info()
    nc, ns = info.num_cores, info.num_subcores
    nw = nc * ns
    rows = b // nw
    wid = lax.axis_index("s") * nc + lax.axis_index("c")
    base = wid * rows
    pltpu.sync_copy(dby_hbm, dby_v)
    pltpu.sync_copy(idx3_hbm.at[wid], idx_v)
    for batch in range(rows // 16):
        cnt = [jnp.zeros((16,), i32) for _ in range(_NCLS)]
        for j in range(_K):
            ii = idx_v[j, pl.ds(batch * 16, 16)]
            labs = plsc.load_gather(dby_v, [ii])
            for cl in range(_NCLS):
                cnt[cl] = cnt[cl] + jnp.where(labs == cl, i32(1), i32(0))
        best_c = cnt[0]
        best_k = jnp.zeros((16,), i32)
        for cl in range(1, _NCLS):
            better = cnt[cl] > best_c
            best_c = jnp.where(better, cnt[cl], best_c)
            best_k = jnp.where(better, jnp.full((16,), cl, i32), best_k)
        pred_v[pl.ds(batch * 16, 16)] = best_k
    pltpu.sync_copy(pred_v, pred_hbm.at[pl.ds(base, rows)])


def kernel(x, conv_w, conv_b, lin_w, lin_b, db_x, db_y):
    b, _ = x.shape
    n_db = db_x.shape[0]
    chunk = _CHUNK
    n_chunks = -(-n_db // chunk)
    n_pad = n_chunks * chunk
    bt = min(_BT, b)

    w1 = conv_w[:, :, 3].T
    b1 = conv_b.reshape(1, -1)
    w2 = lin_w.T
    b2 = lin_b.reshape(1, -1)
    dbt = jnp.pad(-2.0 * db_x, ((0, n_pad - n_db), (0, 0)),
                  constant_values=1e18).T

    f32 = jnp.float32
    i32 = jnp.int32
    grid = (b // bt,)
    outv, outi = pl.pallas_call(
        lambda *a: _knn_kernel(n_chunks, chunk, *a),
        grid=grid,
        in_specs=[
            pl.BlockSpec((bt, x.shape[1]), lambda i: (i, 0)),
            pl.BlockSpec(w1.shape, lambda i: (0, 0)),
            pl.BlockSpec(b1.shape, lambda i: (0, 0)),
            pl.BlockSpec(w2.shape, lambda i: (0, 0)),
            pl.BlockSpec(b2.shape, lambda i: (0, 0)),
            pl.BlockSpec(dbt.shape, lambda i: (0, 0)),
        ],
        out_specs=[
            pl.BlockSpec((bt, _K), lambda i: (i, 0)),
            pl.BlockSpec((bt, _K), lambda i: (i, 0)),
        ],
        out_shape=[
            jax.ShapeDtypeStruct((b, _K), f32),
            jax.ShapeDtypeStruct((b, _K), i32),
        ],
        scratch_shapes=[
            pltpu.VMEM((bt, chunk), f32),
            pltpu.VMEM((bt, _K), f32),
            pltpu.VMEM((bt, _K), i32),
        ],
        compiler_params=pltpu.CompilerParams(
            dimension_semantics=("arbitrary",),
        ),
    )(x, w1, b1, w2, b2, dbt)

    dby = db_y.astype(i32)
    mesh = plsc.VectorSubcoreMesh(core_axis_name="c", subcore_axis_name="s")
    info = plsc.get_sparse_core_info()
    nw = info.num_cores * info.num_subcores
    rows = b // nw
    idx3 = outi.reshape(nw, rows, _K).transpose(0, 2, 1)
    vote = functools.partial(
        pl.kernel,
        mesh=mesh,
        out_type=jax.ShapeDtypeStruct((b,), i32),
        scratch_types=[
            pltpu.VMEM((n_db,), i32),
            pltpu.VMEM((_K, rows), i32),
            pltpu.VMEM((rows,), i32),
        ],
        compiler_params=pltpu.CompilerParams(needs_layout_passes=False),
    )(functools.partial(_vote_kernel, b, n_db))
    pred = vote(idx3, dby)
    return outv, pred
